# baseline (device time: 18941 ns/iter reference)
import jax
import jax.numpy as jnp
from jax import lax
from jax.experimental import pallas as pl
from jax.experimental.pallas import tpu as pltpu

N_DEV = 4
B = 2
SQ = 256
SKV = 256
D_MODEL = 512
H_LOC = 4
DH = 64
CHUNK = H_LOC * DH


def kernel(x, Wq, K_ext, V_ext, Wo):
    def body(x_ref, wq_ref, k_ref, v_ref, wo_ref, out_ref,
             comm_ref, send_sems, recv_sems):
        my_i = lax.axis_index("i")
        left = lax.rem(my_i + N_DEV - 1, N_DEV)
        right = lax.rem(my_i + 1, N_DEV)

        barrier_sem = pltpu.get_barrier_semaphore()
        for nbr in (left, right):
            pl.semaphore_signal(
                barrier_sem, inc=1,
                device_id=(nbr,), device_id_type=pl.DeviceIdType.MESH,
            )
        pl.semaphore_wait(barrier_sem, 2)

        qb = lax.broadcasted_iota(jnp.int32, (SQ, SKV), 0) // 64
        kb = lax.broadcasted_iota(jnp.int32, (SQ, SKV), 1) // 64
        mask = (qb == kb) | (kb == 0) | ((qb + kb) % 3 == 0)

        wq_loc = wq_ref[:, pl.ds(my_i * CHUNK, CHUNK)].astype(jnp.bfloat16)

        def attn_batch(b):
            q_all = jnp.dot(x_ref[b].astype(jnp.bfloat16), wq_loc,
                            preferred_element_type=jnp.float32)
            for h in range(H_LOC):
                q = q_all[:, h * DH:(h + 1) * DH].astype(jnp.bfloat16)
                k = k_ref[b, :, h, :].astype(jnp.bfloat16)
                v = v_ref[b, :, h, :].astype(jnp.bfloat16)
                s = lax.dot_general(
                    q, k, (((1,), (1,)), ((), ())),
                    preferred_element_type=jnp.float32,
                ) * 0.125
                s = jnp.where(mask, s, -1e9)
                m = jnp.max(s, axis=-1, keepdims=True)
                w = jnp.exp(s - m)
                w = w / jnp.sum(w, axis=-1, keepdims=True)
                ctx = jnp.dot(w.astype(jnp.bfloat16), v,
                              preferred_element_type=jnp.float32)
                comm_ref[0, b, :, h * DH:(h + 1) * DH] = ctx.astype(jnp.bfloat16)

        def mk(src, dst, sem, dev):
            return pltpu.make_async_remote_copy(
                src_ref=comm_ref.at[src[0], src[1]],
                dst_ref=comm_ref.at[dst[0], dst[1]],
                send_sem=send_sems.at[sem], recv_sem=recv_sems.at[sem],
                device_id=(dev,), device_id_type=pl.DeviceIdType.MESH,
            )

        attn_batch(0)
        a1b0 = mk((0, 0), (1, 0), 0, right)
        a2b0 = mk((0, 0), (2, 0), 1, left)
        a1b0.start()
        a2b0.start()
        attn_batch(1)
        a1b1 = mk((0, 1), (1, 1), 2, right)
        a2b1 = mk((0, 1), (2, 1), 3, left)
        a1b1.start()
        a2b1.start()

        wo_o = wo_ref[pl.ds(my_i * CHUNK, CHUNK), :].astype(jnp.bfloat16)
        for b in range(B):
            out_ref[b] = jnp.dot(comm_ref[0, b], wo_o,
                                 preferred_element_type=jnp.float32)

        a1b0.wait_recv()
        f1 = mk((1, 0), (3, 0), 4, right)
        f1.start()
        a2b1.wait_recv()
        f2 = mk((2, 1), (3, 1), 5, left)
        f2.start()
        a2b0.wait_recv()
        a1b1.wait_recv()

        for slot, d in ((1, N_DEV - 1), (2, 1)):
            origin = lax.rem(my_i + d, N_DEV)
            wo_o = wo_ref[pl.ds(origin * CHUNK, CHUNK), :].astype(jnp.bfloat16)
            for b in range(B):
                out_ref[b] = out_ref[b] + jnp.dot(
                    comm_ref[slot, b], wo_o,
                    preferred_element_type=jnp.float32)

        for r in (a1b0, a2b0, a1b1, a2b1):
            r.wait_send()
        f1.wait()
        f2.wait()

        origin = lax.rem(my_i + 2, N_DEV)
        wo_o = wo_ref[pl.ds(origin * CHUNK, CHUNK), :].astype(jnp.bfloat16)
        for b in range(B):
            out_ref[b] = out_ref[b] + jnp.dot(
                comm_ref[3, b], wo_o,
                preferred_element_type=jnp.float32)

    return pl.pallas_call(
        body,
        out_shape=jax.ShapeDtypeStruct((B, SQ, D_MODEL), jnp.float32),
        in_specs=[pl.BlockSpec(memory_space=pltpu.VMEM)] * 5,
        out_specs=pl.BlockSpec(memory_space=pltpu.VMEM),
        scratch_shapes=[
            pltpu.VMEM((N_DEV, B, SQ, CHUNK), jnp.bfloat16),
            pltpu.SemaphoreType.DMA((6,)),
            pltpu.SemaphoreType.DMA((6,)),
        ],
        compiler_params=pltpu.CompilerParams(collective_id=0),
    )(x, Wq, K_ext, V_ext, Wo)


# device time: 17908 ns/iter; 1.0577x vs baseline; 1.0577x over previous
import jax
import jax.numpy as jnp
from jax import lax
from jax.experimental import pallas as pl
from jax.experimental.pallas import tpu as pltpu

N_DEV = 4
B = 2
SQ = 256
SKV = 256
D_MODEL = 512
H_LOC = 4
DH = 64
CHUNK = H_LOC * DH


def kernel(x, Wq, K_ext, V_ext, Wo):
    def body(x_ref, wq_ref, k_ref, v_ref, wo_ref, out_ref,
             comm_ref, send_sems, recv_sems):
        my_i = lax.axis_index("i")
        left = lax.rem(my_i + N_DEV - 1, N_DEV)
        right = lax.rem(my_i + 1, N_DEV)

        barrier_sem = pltpu.get_barrier_semaphore()
        for nbr in (left, right):
            pl.semaphore_signal(
                barrier_sem, inc=1,
                device_id=(nbr,), device_id_type=pl.DeviceIdType.MESH,
            )
        pl.semaphore_wait(barrier_sem, 2)

        qb = lax.broadcasted_iota(jnp.int32, (SQ, SKV), 0) // 64
        kb = lax.broadcasted_iota(jnp.int32, (SQ, SKV), 1) // 64
        mask = (qb == kb) | (kb == 0) | ((qb + kb) % 3 == 0)

        wq_loc = wq_ref[:, pl.ds(my_i * CHUNK, CHUNK)]

        def attn_batch(b):
            q_all = jnp.dot(x_ref[b], wq_loc,
                            preferred_element_type=jnp.float32)
            for h in range(H_LOC):
                q = q_all[:, h * DH:(h + 1) * DH]
                k = k_ref[b, :, h, :]
                v = v_ref[b, :, h, :]
                s = lax.dot_general(
                    q, k, (((1,), (1,)), ((), ())),
                    preferred_element_type=jnp.float32,
                ) * 0.125
                s = jnp.where(mask, s, -1e9)
                m = jnp.max(s, axis=-1, keepdims=True)
                w = jnp.exp(s - m)
                w = w / jnp.sum(w, axis=-1, keepdims=True)
                ctx = jnp.dot(w, v, preferred_element_type=jnp.float32)
                comm_ref[0, b, :, h * DH:(h + 1) * DH] = ctx.astype(jnp.bfloat16)

        def mk(src, dst, sem, dev):
            return pltpu.make_async_remote_copy(
                src_ref=comm_ref.at[src[0], src[1]],
                dst_ref=comm_ref.at[dst[0], dst[1]],
                send_sem=send_sems.at[sem], recv_sem=recv_sems.at[sem],
                device_id=(dev,), device_id_type=pl.DeviceIdType.MESH,
            )

        attn_batch(0)
        a1b0 = mk((0, 0), (1, 0), 0, right)
        a2b0 = mk((0, 0), (2, 0), 1, left)
        a1b0.start()
        a2b0.start()
        attn_batch(1)
        a1b1 = mk((0, 1), (1, 1), 2, right)
        a2b1 = mk((0, 1), (2, 1), 3, left)
        a1b1.start()
        a2b1.start()

        wo_o = wo_ref[pl.ds(my_i * CHUNK, CHUNK), :].astype(jnp.bfloat16)
        for b in range(B):
            out_ref[b] = jnp.dot(comm_ref[0, b], wo_o,
                                 preferred_element_type=jnp.float32)

        a1b0.wait_recv()
        f1 = mk((1, 0), (3, 0), 4, right)
        f1.start()
        a2b1.wait_recv()
        f2 = mk((2, 1), (3, 1), 5, left)
        f2.start()
        a2b0.wait_recv()
        a1b1.wait_recv()

        for slot, d in ((1, N_DEV - 1), (2, 1)):
            origin = lax.rem(my_i + d, N_DEV)
            wo_o = wo_ref[pl.ds(origin * CHUNK, CHUNK), :].astype(jnp.bfloat16)
            for b in range(B):
                out_ref[b] = out_ref[b] + jnp.dot(
                    comm_ref[slot, b], wo_o,
                    preferred_element_type=jnp.float32)

        for r in (a1b0, a2b0, a1b1, a2b1):
            r.wait_send()
        f1.wait()
        f2.wait()

        origin = lax.rem(my_i + 2, N_DEV)
        wo_o = wo_ref[pl.ds(origin * CHUNK, CHUNK), :].astype(jnp.bfloat16)
        for b in range(B):
            out_ref[b] = out_ref[b] + jnp.dot(
                comm_ref[3, b], wo_o,
                preferred_element_type=jnp.float32)

    return pl.pallas_call(
        body,
        out_shape=jax.ShapeDtypeStruct((B, SQ, D_MODEL), jnp.float32),
        in_specs=[pl.BlockSpec(memory_space=pltpu.VMEM)] * 5,
        out_specs=pl.BlockSpec(memory_space=pltpu.VMEM),
        scratch_shapes=[
            pltpu.VMEM((N_DEV, B, SQ, CHUNK), jnp.bfloat16),
            pltpu.SemaphoreType.DMA((6,)),
            pltpu.SemaphoreType.DMA((6,)),
        ],
        compiler_params=pltpu.CompilerParams(collective_id=0),
    )(x, Wq, K_ext, V_ext, Wo)
